# Initial kernel scaffold; baseline (speedup 1.0000x reference)
import jax
import jax.numpy as jnp
from jax.experimental import pallas as pl


def kernel(z, edge_index, W_root, W_nbr, b1, W_out, b2):
    raise NotImplementedError("write your pallas kernel here")


# speedup vs baseline: 5.5939x; 5.5939x over previous
import functools

import jax
import jax.numpy as jnp
from jax import lax
from jax.experimental import pallas as pl
from jax.experimental.pallas import tpu as pltpu
from jax.experimental.pallas import tpu_sc as plsc

N = 10000
E = 320000
D = 128
DH = D // 2

NC = 2
NS = 16
EPT = E // NS
K = 80
NCHUNK = EPT // K
ROWS_PER_TILE = 640
NPAD = ROWS_PER_TILE * NS


def _sc_segment_sum(z_lo, z_hi, src3, dst3, zeros):
  mesh = plsc.VectorSubcoreMesh(core_axis_name="c", subcore_axis_name="s")

  @functools.partial(
      pl.kernel,
      mesh=mesh,
      compiler_params=pltpu.CompilerParams(use_tc_tiling_on_sc=False),
      out_type=jax.ShapeDtypeStruct((NC, NPAD, DH), jnp.float32),
      scratch_types=[
          pltpu.VMEM((NCHUNK, K), jnp.int32),
          pltpu.VMEM((NCHUNK, K), jnp.int32),
          pltpu.VMEM((K, DH), jnp.float32),
          pltpu.VMEM_SHARED((NPAD, DH), jnp.float32),
          pltpu.SemaphoreType.DMA,
      ],
  )
  def k(zlo_hbm, zhi_hbm, src_hbm, dst_hbm, zeros_hbm, out_hbm,
        src_v, dst_v, rows_v, acc, sem):
    c = lax.axis_index("c")
    s = lax.axis_index("s")

    r0 = s * ROWS_PER_TILE
    pltpu.sync_copy(zeros_hbm.at[pl.ds(r0, ROWS_PER_TILE)],
                    acc.at[pl.ds(r0, ROWS_PER_TILE)])
    pltpu.sync_copy(src_hbm.at[s], src_v)
    pltpu.sync_copy(dst_hbm.at[s], dst_v)
    plsc.subcore_barrier()

    def body_lo(j, carry):
      pltpu.async_copy(zlo_hbm.at[src_v.at[j]], rows_v, sem).wait()
      pltpu.sync_copy(rows_v, acc.at[dst_v.at[j]], add=True)
      return carry

    def body_hi(j, carry):
      pltpu.async_copy(zhi_hbm.at[src_v.at[j]], rows_v, sem).wait()
      pltpu.sync_copy(rows_v, acc.at[dst_v.at[j]], add=True)
      return carry

    @pl.when(c == 0)
    def _():
      lax.fori_loop(0, NCHUNK, body_lo, 0)

    @pl.when(c == 1)
    def _():
      lax.fori_loop(0, NCHUNK, body_hi, 0)

    plsc.subcore_barrier()

    pltpu.sync_copy(acc.at[pl.ds(r0, ROWS_PER_TILE)],
                    out_hbm.at[c].at[pl.ds(r0, ROWS_PER_TILE)])

  return k(z_lo, z_hi, src3, dst3, zeros)


def _tc_body(p_ref, z_ref, wr_ref, wnl_ref, wnh_ref, b1_ref, wo_ref, b2_ref,
             o_ref):
  h = jnp.dot(z_ref[...], wr_ref[...], preferred_element_type=jnp.float32)
  h = h + jnp.dot(p_ref[0], wnl_ref[...], preferred_element_type=jnp.float32)
  h = h + jnp.dot(p_ref[1], wnh_ref[...], preferred_element_type=jnp.float32)
  h = jnp.maximum(h + b1_ref[...], 0.0)
  o_ref[...] = jnp.dot(h, wo_ref[...], preferred_element_type=jnp.float32) + b2_ref[...]


def _tc_decoder(partial, z, W_root, W_nbr, b1, W_out, b2):
  BN = 1000
  return pl.pallas_call(
      _tc_body,
      grid=(N // BN,),
      in_specs=[
          pl.BlockSpec((NC, BN, DH), lambda i: (0, i, 0)),
          pl.BlockSpec((BN, D), lambda i: (i, 0)),
          pl.BlockSpec((D, D), lambda i: (0, 0)),
          pl.BlockSpec((DH, D), lambda i: (0, 0)),
          pl.BlockSpec((DH, D), lambda i: (1, 0)),
          pl.BlockSpec((1, D), lambda i: (0, 0)),
          pl.BlockSpec((D, D), lambda i: (0, 0)),
          pl.BlockSpec((1, D), lambda i: (0, 0)),
      ],
      out_specs=pl.BlockSpec((BN, D), lambda i: (i, 0)),
      out_shape=jax.ShapeDtypeStruct((N, D), jnp.float32),
  )(partial, z, W_root, W_nbr, W_nbr, b1, W_out, b2)


def kernel(z, edge_index, W_root, W_nbr, b1, W_out, b2):
  z_lo = z[:, :DH]
  z_hi = z[:, DH:]
  src3 = edge_index[0].reshape(NS, NCHUNK, K)
  dst3 = edge_index[1].reshape(NS, NCHUNK, K)
  zeros = jnp.zeros((NPAD, DH), jnp.float32)
  partial = _sc_segment_sum(z_lo, z_hi, src3, dst3, zeros)
  return _tc_decoder(partial, z, W_root, W_nbr,
                     b1.reshape(1, D), W_out, b2.reshape(1, D))


# speedup vs baseline: 8.8815x; 1.5877x over previous
import functools

import jax
import jax.numpy as jnp
from jax import lax
from jax.experimental import pallas as pl
from jax.experimental.pallas import tpu as pltpu
from jax.experimental.pallas import tpu_sc as plsc

N = 10000
E = 320000
D = 128
DH = D // 2

NC = 2
NS = 16
EPT = E // NS
K = 80
NCHUNK = EPT // K
ROWS_PER_TILE = 640
NPAD = ROWS_PER_TILE * NS


def _sc_segment_sum(z_lo, z_hi, src3, dst3, zeros):
  mesh = plsc.VectorSubcoreMesh(core_axis_name="c", subcore_axis_name="s")

  @functools.partial(
      pl.kernel,
      mesh=mesh,
      compiler_params=pltpu.CompilerParams(use_tc_tiling_on_sc=False),
      out_type=jax.ShapeDtypeStruct((NC, NPAD, DH), jnp.float32),
      scratch_types=[
          pltpu.VMEM((NCHUNK, K), jnp.int32),
          pltpu.VMEM((NCHUNK, K), jnp.int32),
          pltpu.VMEM((K, DH), jnp.float32),
          pltpu.VMEM((K, DH), jnp.float32),
          pltpu.VMEM_SHARED((NPAD, DH), jnp.float32),
          pltpu.SemaphoreType.DMA,
          pltpu.SemaphoreType.DMA,
      ],
  )
  def k(zlo_hbm, zhi_hbm, src_hbm, dst_hbm, zeros_hbm, out_hbm,
        src_v, dst_v, rows0_v, rows1_v, acc, sem0, sem1):
    c = lax.axis_index("c")
    s = lax.axis_index("s")

    r0 = s * ROWS_PER_TILE
    pltpu.sync_copy(zeros_hbm.at[pl.ds(r0, ROWS_PER_TILE)],
                    acc.at[pl.ds(r0, ROWS_PER_TILE)])
    pltpu.sync_copy(src_hbm.at[s], src_v)
    pltpu.sync_copy(dst_hbm.at[s], dst_v)
    plsc.subcore_barrier()

    def run(tab):
      pltpu.async_copy(tab.at[src_v.at[0]], rows0_v, sem0)

      def body2(i, carry):
        j0 = 2 * i
        pltpu.async_copy(tab.at[src_v.at[j0 + 1]], rows1_v, sem1)
        pltpu.make_async_copy(tab.at[src_v.at[j0]], rows0_v, sem0).wait()
        pltpu.sync_copy(rows0_v, acc.at[dst_v.at[j0]], add=True)

        @pl.when(j0 + 2 < NCHUNK)
        def _():
          pltpu.async_copy(tab.at[src_v.at[j0 + 2]], rows0_v, sem0)

        pltpu.make_async_copy(tab.at[src_v.at[j0 + 1]], rows1_v, sem1).wait()
        pltpu.sync_copy(rows1_v, acc.at[dst_v.at[j0 + 1]], add=True)
        return carry

      lax.fori_loop(0, NCHUNK // 2, body2, 0)

    @pl.when(c == 0)
    def _():
      run(zlo_hbm)

    @pl.when(c == 1)
    def _():
      run(zhi_hbm)

    plsc.subcore_barrier()

    pltpu.sync_copy(acc.at[pl.ds(r0, ROWS_PER_TILE)],
                    out_hbm.at[c].at[pl.ds(r0, ROWS_PER_TILE)])

  return k(z_lo, z_hi, src3, dst3, zeros)


def _tc_body(p_ref, z_ref, wr_ref, wnl_ref, wnh_ref, b1_ref, wo_ref, b2_ref,
             o_ref):
  h = jnp.dot(z_ref[...], wr_ref[...], preferred_element_type=jnp.float32)
  h = h + jnp.dot(p_ref[0], wnl_ref[...], preferred_element_type=jnp.float32)
  h = h + jnp.dot(p_ref[1], wnh_ref[...], preferred_element_type=jnp.float32)
  h = jnp.maximum(h + b1_ref[...], 0.0)
  o_ref[...] = jnp.dot(h, wo_ref[...], preferred_element_type=jnp.float32) + b2_ref[...]


def _tc_decoder(partial, z, W_root, W_nbr, b1, W_out, b2):
  BN = 1000
  return pl.pallas_call(
      _tc_body,
      grid=(N // BN,),
      in_specs=[
          pl.BlockSpec((NC, BN, DH), lambda i: (0, i, 0)),
          pl.BlockSpec((BN, D), lambda i: (i, 0)),
          pl.BlockSpec((D, D), lambda i: (0, 0)),
          pl.BlockSpec((DH, D), lambda i: (0, 0)),
          pl.BlockSpec((DH, D), lambda i: (1, 0)),
          pl.BlockSpec((1, D), lambda i: (0, 0)),
          pl.BlockSpec((D, D), lambda i: (0, 0)),
          pl.BlockSpec((1, D), lambda i: (0, 0)),
      ],
      out_specs=pl.BlockSpec((BN, D), lambda i: (i, 0)),
      out_shape=jax.ShapeDtypeStruct((N, D), jnp.float32),
  )(partial, z, W_root, W_nbr, W_nbr, b1, W_out, b2)


def kernel(z, edge_index, W_root, W_nbr, b1, W_out, b2):
  z_lo = z[:, :DH]
  z_hi = z[:, DH:]
  src3 = edge_index[0].reshape(NS, NCHUNK, K)
  dst3 = edge_index[1].reshape(NS, NCHUNK, K)
  zeros = jnp.zeros((NPAD, DH), jnp.float32)
  partial = _sc_segment_sum(z_lo, z_hi, src3, dst3, zeros)
  return _tc_decoder(partial, z, W_root, W_nbr,
                     b1.reshape(1, D), W_out, b2.reshape(1, D))


# speedup vs baseline: 11.8372x; 1.3328x over previous
import functools

import jax
import jax.numpy as jnp
from jax import lax
from jax.experimental import pallas as pl
from jax.experimental.pallas import tpu as pltpu
from jax.experimental.pallas import tpu_sc as plsc

N = 10000
E = 320000
D = 128
DH = D // 2

NC = 2
NS = 16
EPT = E // NS
K = 125
NCHUNK = EPT // K
NBUF = 4
ROWS_PER_TILE = 640
NPAD = ROWS_PER_TILE * NS


def _sc_segment_sum(z_lo, z_hi, src3, dst3, zeros):
  mesh = plsc.VectorSubcoreMesh(core_axis_name="c", subcore_axis_name="s")

  @functools.partial(
      pl.kernel,
      mesh=mesh,
      compiler_params=pltpu.CompilerParams(use_tc_tiling_on_sc=False),
      out_type=jax.ShapeDtypeStruct((NC, NPAD, DH), jnp.float32),
      scratch_types=[
          pltpu.VMEM((NCHUNK, K), jnp.int32),
          pltpu.VMEM((NCHUNK, K), jnp.int32),
          [pltpu.VMEM((K, DH), jnp.float32)] * NBUF,
          pltpu.VMEM_SHARED((NPAD, DH), jnp.float32),
          [pltpu.SemaphoreType.DMA] * NBUF,
          [pltpu.SemaphoreType.DMA] * NBUF,
      ],
  )
  def k(zlo_hbm, zhi_hbm, src_hbm, dst_hbm, zeros_hbm, out_hbm,
        src_v, dst_v, rows, acc, gsem, ssem):
    c = lax.axis_index("c")
    s = lax.axis_index("s")

    r0 = s * ROWS_PER_TILE
    pltpu.sync_copy(zeros_hbm.at[pl.ds(r0, ROWS_PER_TILE)],
                    acc.at[pl.ds(r0, ROWS_PER_TILE)])
    pltpu.sync_copy(src_hbm.at[s], src_v)
    pltpu.sync_copy(dst_hbm.at[s], dst_v)
    plsc.subcore_barrier()

    def run(tab):
      def wait_gather(j, v):
        pltpu.make_async_copy(tab.at[src_v.at[j]], rows[v], gsem[v]).wait()

      def start_scatter(j, v):
        pltpu.async_copy(rows[v], acc.at[dst_v.at[j]], ssem[v], add=True)

      def wait_scatter(j, v):
        pltpu.make_async_copy(rows[v], acc.at[dst_v.at[j]], ssem[v]).wait()

      def body(i, carry):
        j0 = NBUF * i
        for v in range(NBUF):
          j = j0 + v

          @pl.when(j >= NBUF)
          def _():
            wait_scatter(j - NBUF, v)

          pltpu.async_copy(tab.at[src_v.at[j]], rows[v], gsem[v])

          @pl.when(j >= NBUF - 1)
          def _():
            jl = j - (NBUF - 1)
            wait_gather(jl, (v + 1) % NBUF)
            start_scatter(jl, (v + 1) % NBUF)

        return carry

      lax.fori_loop(0, NCHUNK // NBUF, body, 0)
      wait_scatter(NCHUNK - NBUF, (NCHUNK - NBUF) % NBUF)
      for r in range(NCHUNK - NBUF + 1, NCHUNK):
        v = r % NBUF
        wait_gather(r, v)
        pltpu.sync_copy(rows[v], acc.at[dst_v.at[r]], add=True)

    @pl.when(c == 0)
    def _():
      run(zlo_hbm)

    @pl.when(c == 1)
    def _():
      run(zhi_hbm)

    plsc.subcore_barrier()

    pltpu.sync_copy(acc.at[pl.ds(r0, ROWS_PER_TILE)],
                    out_hbm.at[c].at[pl.ds(r0, ROWS_PER_TILE)])

  return k(z_lo, z_hi, src3, dst3, zeros)


def _tc_body(p_ref, z_ref, wr_ref, wnl_ref, wnh_ref, b1_ref, wo_ref, b2_ref,
             o_ref):
  h = jnp.dot(z_ref[...], wr_ref[...], preferred_element_type=jnp.float32)
  h = h + jnp.dot(p_ref[0], wnl_ref[...], preferred_element_type=jnp.float32)
  h = h + jnp.dot(p_ref[1], wnh_ref[...], preferred_element_type=jnp.float32)
  h = jnp.maximum(h + b1_ref[...], 0.0)
  o_ref[...] = jnp.dot(h, wo_ref[...], preferred_element_type=jnp.float32) + b2_ref[...]


def _tc_decoder(partial, z, W_root, W_nbr, b1, W_out, b2):
  BN = 1000
  return pl.pallas_call(
      _tc_body,
      grid=(N // BN,),
      in_specs=[
          pl.BlockSpec((NC, BN, DH), lambda i: (0, i, 0)),
          pl.BlockSpec((BN, D), lambda i: (i, 0)),
          pl.BlockSpec((D, D), lambda i: (0, 0)),
          pl.BlockSpec((DH, D), lambda i: (0, 0)),
          pl.BlockSpec((DH, D), lambda i: (1, 0)),
          pl.BlockSpec((1, D), lambda i: (0, 0)),
          pl.BlockSpec((D, D), lambda i: (0, 0)),
          pl.BlockSpec((1, D), lambda i: (0, 0)),
      ],
      out_specs=pl.BlockSpec((BN, D), lambda i: (i, 0)),
      out_shape=jax.ShapeDtypeStruct((N, D), jnp.float32),
  )(partial, z, W_root, W_nbr, W_nbr, b1, W_out, b2)


def kernel(z, edge_index, W_root, W_nbr, b1, W_out, b2):
  z_lo = z[:, :DH]
  z_hi = z[:, DH:]
  src3 = edge_index[0].reshape(NS, NCHUNK, K)
  dst3 = edge_index[1].reshape(NS, NCHUNK, K)
  zeros = jnp.zeros((NPAD, DH), jnp.float32)
  partial = _sc_segment_sum(z_lo, z_hi, src3, dst3, zeros)
  return _tc_decoder(partial, z, W_root, W_nbr,
                     b1.reshape(1, D), W_out, b2.reshape(1, D))


# speedup vs baseline: 13.4234x; 1.1340x over previous
import functools

import jax
import jax.numpy as jnp
from jax import lax
from jax.experimental import pallas as pl
from jax.experimental.pallas import tpu as pltpu
from jax.experimental.pallas import tpu_sc as plsc

N = 10000
E = 320000
D = 128
DH = D // 2

NC = 2
NS = 16
EPT = E // NS
K = 125
NCHUNK = EPT // K
NBUF = 4
ROWS_PER_TILE = 640
NPAD = ROWS_PER_TILE * NS


def _sc_segment_sum(z2, idx4, zeros):
  mesh = plsc.VectorSubcoreMesh(core_axis_name="c", subcore_axis_name="s")

  @functools.partial(
      pl.kernel,
      mesh=mesh,
      compiler_params=pltpu.CompilerParams(use_tc_tiling_on_sc=False),
      out_type=jax.ShapeDtypeStruct((NPAD, D), jnp.float32),
      scratch_types=[
          pltpu.VMEM((NCHUNK, K), jnp.int32),
          pltpu.VMEM((NCHUNK, K), jnp.int32),
          [pltpu.VMEM((K, DH), jnp.float32)] * NBUF,
          pltpu.VMEM_SHARED((NPAD, DH), jnp.float32),
          [pltpu.SemaphoreType.DMA] * NBUF,
          [pltpu.SemaphoreType.DMA] * NBUF,
      ],
  )
  def k(z2_hbm, idx_hbm, zeros_hbm, out_hbm, src_v, dst_v, rows, acc,
        gsem, ssem):
    c = lax.axis_index("c")
    s = lax.axis_index("s")

    r0 = s * ROWS_PER_TILE
    pltpu.sync_copy(zeros_hbm.at[pl.ds(r0, ROWS_PER_TILE)],
                    acc.at[pl.ds(r0, ROWS_PER_TILE)])
    pltpu.sync_copy(idx_hbm.at[c].at[s], src_v)
    pltpu.sync_copy(idx_hbm.at[2].at[s], dst_v)
    plsc.subcore_barrier()

    def wait_gather(j, v):
      pltpu.make_async_copy(z2_hbm.at[src_v.at[j]], rows[v], gsem[v]).wait()

    def start_scatter(j, v):
      pltpu.async_copy(rows[v], acc.at[dst_v.at[j]], ssem[v], add=True)

    def wait_scatter(j, v):
      pltpu.make_async_copy(rows[v], acc.at[dst_v.at[j]], ssem[v]).wait()

    def body(i, carry):
      j0 = NBUF * i
      for v in range(NBUF):
        j = j0 + v

        @pl.when(j >= NBUF)
        def _():
          wait_scatter(j - NBUF, v)

        pltpu.async_copy(z2_hbm.at[src_v.at[j]], rows[v], gsem[v])

        @pl.when(j >= NBUF - 1)
        def _():
          jl = j - (NBUF - 1)
          wait_gather(jl, (v + 1) % NBUF)
          start_scatter(jl, (v + 1) % NBUF)

      return carry

    lax.fori_loop(0, NCHUNK // NBUF, body, 0)
    wait_scatter(NCHUNK - NBUF, (NCHUNK - NBUF) % NBUF)
    for r in range(NCHUNK - NBUF + 1, NCHUNK):
      v = r % NBUF
      wait_gather(r, v)
      pltpu.sync_copy(rows[v], acc.at[dst_v.at[r]], add=True)

    plsc.subcore_barrier()

    pltpu.sync_copy(acc.at[pl.ds(r0, ROWS_PER_TILE)],
                    out_hbm.at[pl.ds(r0, ROWS_PER_TILE), pl.ds(c * DH, DH)])

  return k(z2, idx4, zeros)


def _tc_body(agg_ref, z_ref, wr_ref, wn_ref, b1_ref, wo_ref, b2_ref, o_ref):
  h = jnp.dot(z_ref[...], wr_ref[...], preferred_element_type=jnp.float32)
  h = h + jnp.dot(agg_ref[...], wn_ref[...], preferred_element_type=jnp.float32)
  h = jnp.maximum(h + b1_ref[...], 0.0)
  o_ref[...] = jnp.dot(h, wo_ref[...], preferred_element_type=jnp.float32) + b2_ref[...]


def _tc_decoder(agg, z, W_root, W_nbr, b1, W_out, b2):
  BN = 1000
  return pl.pallas_call(
      _tc_body,
      grid=(N // BN,),
      in_specs=[
          pl.BlockSpec((BN, D), lambda i: (i, 0)),
          pl.BlockSpec((BN, D), lambda i: (i, 0)),
          pl.BlockSpec((D, D), lambda i: (0, 0)),
          pl.BlockSpec((D, D), lambda i: (0, 0)),
          pl.BlockSpec((1, D), lambda i: (0, 0)),
          pl.BlockSpec((D, D), lambda i: (0, 0)),
          pl.BlockSpec((1, D), lambda i: (0, 0)),
      ],
      out_specs=pl.BlockSpec((BN, D), lambda i: (i, 0)),
      out_shape=jax.ShapeDtypeStruct((N, D), jnp.float32),
  )(agg, z, W_root, W_nbr, b1, W_out, b2)


def kernel(z, edge_index, W_root, W_nbr, b1, W_out, b2):
  base = edge_index[0] * 2
  idx4 = jnp.stack([base, base + 1, edge_index[1]], 0).reshape(
      3, NS, NCHUNK, K)
  z2 = z.reshape(2 * N, DH)
  zeros = jnp.zeros((NPAD, DH), jnp.float32)
  agg = _sc_segment_sum(z2, idx4, zeros)
  return _tc_decoder(agg, z, W_root, W_nbr,
                     b1.reshape(1, D), W_out, b2.reshape(1, D))


# speedup vs baseline: 15.4715x; 1.1526x over previous
import functools

import jax
import jax.numpy as jnp
from jax import lax
from jax.experimental import pallas as pl
from jax.experimental.pallas import tpu as pltpu
from jax.experimental.pallas import tpu_sc as plsc

N = 10000
E = 320000
D = 128
DH = D // 2

NC = 2
NS = 16
EPT = E // NS
K = 128
NCHUNK = EPT // K
KTAIL = EPT - NCHUNK * K
NBUF = 4
ROWS_PER_TILE = 640
NPAD = ROWS_PER_TILE * NS
VL = 16


def _sc_segment_sum(z2, edge_index, zeros):
  mesh = plsc.VectorSubcoreMesh(core_axis_name="c", subcore_axis_name="s")

  @functools.partial(
      pl.kernel,
      mesh=mesh,
      compiler_params=pltpu.CompilerParams(use_tc_tiling_on_sc=False),
      out_type=jax.ShapeDtypeStruct((NPAD, D), jnp.float32),
      scratch_types=[
          pltpu.VMEM((EPT,), jnp.int32),
          pltpu.VMEM((EPT,), jnp.int32),
          [pltpu.VMEM((K, DH), jnp.float32)] * NBUF,
          pltpu.VMEM_SHARED((NPAD, DH), jnp.float32),
          [pltpu.SemaphoreType.DMA] * NBUF,
          [pltpu.SemaphoreType.DMA] * NBUF,
      ],
  )
  def k(z2_hbm, ei_hbm, zeros_hbm, out_hbm, src_v, dst_v, rows, acc,
        gsem, ssem):
    c = lax.axis_index("c")
    s = lax.axis_index("s")

    r0 = s * ROWS_PER_TILE
    pltpu.sync_copy(zeros_hbm.at[pl.ds(r0, ROWS_PER_TILE)],
                    acc.at[pl.ds(r0, ROWS_PER_TILE)])
    pltpu.sync_copy(ei_hbm.at[0, pl.ds(s * EPT, EPT)], src_v)
    pltpu.sync_copy(ei_hbm.at[1, pl.ds(s * EPT, EPT)], dst_v)

    def tbody(i, carry):
      for u in range(10):
        off = i * (10 * VL) + u * VL
        src_v[pl.ds(off, VL)] = src_v[pl.ds(off, VL)] * 2 + c
      return carry

    lax.fori_loop(0, EPT // (10 * VL), tbody, 0)
    plsc.subcore_barrier()

    def gidx(j):
      return src_v.at[pl.ds(j * K, K)]

    def didx(j):
      return dst_v.at[pl.ds(j * K, K)]

    def wait_gather(j, v):
      pltpu.make_async_copy(z2_hbm.at[gidx(j)], rows[v], gsem[v]).wait()

    def start_scatter(j, v):
      pltpu.async_copy(rows[v], acc.at[didx(j)], ssem[v], add=True)

    def wait_scatter(j, v):
      pltpu.make_async_copy(rows[v], acc.at[didx(j)], ssem[v]).wait()

    def body(i, carry):
      j0 = NBUF * i
      for v in range(NBUF):
        j = j0 + v

        @pl.when(j >= NBUF)
        def _():
          wait_scatter(j - NBUF, v)

        pltpu.async_copy(z2_hbm.at[gidx(j)], rows[v], gsem[v])

        @pl.when(j >= NBUF - 1)
        def _():
          jl = j - (NBUF - 1)
          wait_gather(jl, (v + 1) % NBUF)
          start_scatter(jl, (v + 1) % NBUF)

      return carry

    lax.fori_loop(0, NCHUNK // NBUF, body, 0)
    wait_scatter(NCHUNK - NBUF, (NCHUNK - NBUF) % NBUF)
    for r in range(NCHUNK - NBUF + 1, NCHUNK):
      v = r % NBUF
      wait_gather(r, v)
      pltpu.sync_copy(rows[v], acc.at[didx(r)], add=True)

    toff = NCHUNK * K
    tsrc = src_v.at[pl.ds(toff, KTAIL)]
    tdst = dst_v.at[pl.ds(toff, KTAIL)]
    trows = rows[0].at[pl.ds(0, KTAIL)]
    pltpu.async_copy(z2_hbm.at[tsrc], trows, gsem[0]).wait()
    pltpu.sync_copy(trows, acc.at[tdst], add=True)

    plsc.subcore_barrier()

    pltpu.sync_copy(acc.at[pl.ds(r0, ROWS_PER_TILE)],
                    out_hbm.at[pl.ds(r0, ROWS_PER_TILE), pl.ds(c * DH, DH)])

  return k(z2, edge_index, zeros)


def _tc_body(agg_ref, z_ref, wr_ref, wn_ref, b1_ref, wo_ref, b2_ref, o_ref):
  h = jnp.dot(z_ref[...], wr_ref[...], preferred_element_type=jnp.float32)
  h = h + jnp.dot(agg_ref[...], wn_ref[...], preferred_element_type=jnp.float32)
  h = jnp.maximum(h + b1_ref[...], 0.0)
  o_ref[...] = jnp.dot(h, wo_ref[...], preferred_element_type=jnp.float32) + b2_ref[...]


def _tc_decoder(agg, z, W_root, W_nbr, b1, W_out, b2):
  BN = 1000
  return pl.pallas_call(
      _tc_body,
      grid=(N // BN,),
      in_specs=[
          pl.BlockSpec((BN, D), lambda i: (i, 0)),
          pl.BlockSpec((BN, D), lambda i: (i, 0)),
          pl.BlockSpec((D, D), lambda i: (0, 0)),
          pl.BlockSpec((D, D), lambda i: (0, 0)),
          pl.BlockSpec((1, D), lambda i: (0, 0)),
          pl.BlockSpec((D, D), lambda i: (0, 0)),
          pl.BlockSpec((1, D), lambda i: (0, 0)),
      ],
      out_specs=pl.BlockSpec((BN, D), lambda i: (i, 0)),
      out_shape=jax.ShapeDtypeStruct((N, D), jnp.float32),
  )(agg, z, W_root, W_nbr, b1, W_out, b2)


def kernel(z, edge_index, W_root, W_nbr, b1, W_out, b2):
  z2 = z.reshape(2 * N, DH)
  zeros = jnp.zeros((NPAD, DH), jnp.float32)
  agg = _sc_segment_sum(z2, edge_index, zeros)
  return _tc_decoder(agg, z, W_root, W_nbr,
                     b1.reshape(1, D), W_out, b2.reshape(1, D))


# speedup vs baseline: 15.8887x; 1.0270x over previous
import functools

import jax
import jax.numpy as jnp
from jax import lax
from jax.experimental import pallas as pl
from jax.experimental.pallas import tpu as pltpu
from jax.experimental.pallas import tpu_sc as plsc

N = 10000
E = 320000
D = 128
DH = D // 2

NC = 2
NS = 16
EPT = E // NS
K = 128
NCHUNK = EPT // K
KTAIL = EPT - NCHUNK * K
NBUF = 4
ROWS_PER_TILE = 640
NPAD = ROWS_PER_TILE * NS
VL = 16


def _sc_segment_sum(z2, edge_index, zeros):
  mesh = plsc.VectorSubcoreMesh(core_axis_name="c", subcore_axis_name="s")

  @functools.partial(
      pl.kernel,
      mesh=mesh,
      compiler_params=pltpu.CompilerParams(use_tc_tiling_on_sc=False),
      out_type=jax.ShapeDtypeStruct((NPAD, D), jnp.float32),
      scratch_types=[
          pltpu.VMEM((EPT + 3 * K - EPT % K,), jnp.int32),
          pltpu.VMEM((EPT,), jnp.int32),
          [pltpu.VMEM((K, DH), jnp.float32)] * NBUF,
          pltpu.VMEM_SHARED((NPAD, DH), jnp.float32),
          [pltpu.SemaphoreType.DMA] * NBUF,
          [pltpu.SemaphoreType.DMA] * NBUF,
      ],
  )
  def k(z2_hbm, ei_hbm, zeros_hbm, out_hbm, src_v, dst_v, rows, acc,
        gsem, ssem):
    c = lax.axis_index("c")
    s = lax.axis_index("s")

    r0 = s * ROWS_PER_TILE
    pltpu.sync_copy(zeros_hbm.at[pl.ds(r0, ROWS_PER_TILE)],
                    acc.at[pl.ds(r0, ROWS_PER_TILE)])
    pltpu.sync_copy(ei_hbm.at[0, pl.ds(s * EPT, EPT)],
                    src_v.at[pl.ds(0, EPT)])
    pltpu.sync_copy(ei_hbm.at[1, pl.ds(s * EPT, EPT)], dst_v)

    def transform(j):
      for u in range(K // VL):
        off = j * K + u * VL
        src_v[pl.ds(off, VL)] = src_v[pl.ds(off, VL)] * 2 + c

    transform(0)
    plsc.subcore_barrier()

    def gidx(j):
      return src_v.at[pl.ds(j * K, K)]

    def didx(j):
      return dst_v.at[pl.ds(j * K, K)]

    def wait_gather(j, v):
      pltpu.make_async_copy(z2_hbm.at[gidx(j)], rows[v], gsem[v]).wait()

    def start_scatter(j, v):
      pltpu.async_copy(rows[v], acc.at[didx(j)], ssem[v], add=True)

    def wait_scatter(j, v):
      pltpu.make_async_copy(rows[v], acc.at[didx(j)], ssem[v]).wait()

    def body(i, carry):
      j0 = NBUF * i
      for v in range(NBUF):
        j = j0 + v

        @pl.when(j >= NBUF)
        def _():
          wait_scatter(j - NBUF, v)

        pltpu.async_copy(z2_hbm.at[gidx(j)], rows[v], gsem[v])
        transform(j + 1)

        @pl.when(j >= NBUF - 1)
        def _():
          jl = j - (NBUF - 1)
          wait_gather(jl, (v + 1) % NBUF)
          start_scatter(jl, (v + 1) % NBUF)

      return carry

    lax.fori_loop(0, NCHUNK // NBUF, body, 0)
    wait_scatter(NCHUNK - NBUF, (NCHUNK - NBUF) % NBUF)
    for r in range(NCHUNK - NBUF + 1, NCHUNK):
      v = r % NBUF
      wait_gather(r, v)
      pltpu.sync_copy(rows[v], acc.at[didx(r)], add=True)

    toff = NCHUNK * K
    tsrc = src_v.at[pl.ds(toff, KTAIL)]
    tdst = dst_v.at[pl.ds(toff, KTAIL)]
    trows = rows[0].at[pl.ds(0, KTAIL)]
    pltpu.async_copy(z2_hbm.at[tsrc], trows, gsem[0]).wait()
    pltpu.sync_copy(trows, acc.at[tdst], add=True)

    plsc.subcore_barrier()

    pltpu.sync_copy(acc.at[pl.ds(r0, ROWS_PER_TILE)],
                    out_hbm.at[pl.ds(r0, ROWS_PER_TILE), pl.ds(c * DH, DH)])

  return k(z2, edge_index, zeros)


def _tc_zr_body(z_ref, wr_ref, b1_ref, o_ref):
  o_ref[...] = (jnp.dot(z_ref[...], wr_ref[...],
                        preferred_element_type=jnp.float32) + b1_ref[...])


def _tc_zr(z, W_root, b1):
  BN = 2000
  return pl.pallas_call(
      _tc_zr_body,
      grid=(N // BN,),
      in_specs=[
          pl.BlockSpec((BN, D), lambda i: (i, 0)),
          pl.BlockSpec((D, D), lambda i: (0, 0)),
          pl.BlockSpec((1, D), lambda i: (0, 0)),
      ],
      out_specs=pl.BlockSpec((BN, D), lambda i: (i, 0)),
      out_shape=jax.ShapeDtypeStruct((N, D), jnp.float32),
  )(z, W_root, b1)


def _tc_body(agg_ref, zr_ref, wn_ref, wo_ref, b2_ref, o_ref):
  h = zr_ref[...] + jnp.dot(agg_ref[...], wn_ref[...],
                            preferred_element_type=jnp.float32)
  h = jnp.maximum(h, 0.0)
  o_ref[...] = jnp.dot(h, wo_ref[...], preferred_element_type=jnp.float32) + b2_ref[...]


def _tc_decoder(agg, zr, W_nbr, W_out, b2):
  BN = 2000
  return pl.pallas_call(
      _tc_body,
      grid=(N // BN,),
      in_specs=[
          pl.BlockSpec((BN, D), lambda i: (i, 0)),
          pl.BlockSpec((BN, D), lambda i: (i, 0)),
          pl.BlockSpec((D, D), lambda i: (0, 0)),
          pl.BlockSpec((D, D), lambda i: (0, 0)),
          pl.BlockSpec((1, D), lambda i: (0, 0)),
      ],
      out_specs=pl.BlockSpec((BN, D), lambda i: (i, 0)),
      out_shape=jax.ShapeDtypeStruct((N, D), jnp.float32),
  )(agg, zr, W_nbr, W_out, b2)


def kernel(z, edge_index, W_root, W_nbr, b1, W_out, b2):
  z2 = z.reshape(2 * N, DH)
  zeros = jnp.zeros((NPAD, DH), jnp.float32)
  zr = _tc_zr(z, W_root, b1.reshape(1, D))
  agg = _sc_segment_sum(z2, edge_index, zeros)
  return _tc_decoder(agg, zr, W_nbr, W_out, b2.reshape(1, D))


# speedup vs baseline: 16.1938x; 1.0192x over previous
import functools

import jax
import jax.numpy as jnp
from jax import lax
from jax.experimental import pallas as pl
from jax.experimental.pallas import tpu as pltpu
from jax.experimental.pallas import tpu_sc as plsc

N = 10000
E = 320000
D = 128
DH = D // 2

NC = 2
NS = 16
EPT = E // NS
K = 128
NCHUNK = EPT // K
KTAIL = EPT - NCHUNK * K
NBUF = 4
ROWS_PER_TILE = 640
NPAD = ROWS_PER_TILE * NS
VL = 16


def _sc_segment_sum(z2, edge_index):
  mesh = plsc.VectorSubcoreMesh(core_axis_name="c", subcore_axis_name="s")

  @functools.partial(
      pl.kernel,
      mesh=mesh,
      compiler_params=pltpu.CompilerParams(use_tc_tiling_on_sc=False),
      out_type=jax.ShapeDtypeStruct((NPAD, D), jnp.float32),
      scratch_types=[
          pltpu.VMEM((EPT + 3 * K - EPT % K,), jnp.int32),
          pltpu.VMEM((EPT,), jnp.int32),
          [pltpu.VMEM((K, DH), jnp.float32)] * NBUF,
          pltpu.VMEM_SHARED((NPAD, DH), jnp.float32),
          [pltpu.SemaphoreType.DMA] * NBUF,
          [pltpu.SemaphoreType.DMA] * NBUF,
      ],
  )
  def k(z2_hbm, ei_hbm, out_hbm, src_v, dst_v, rows, acc, gsem, ssem):
    c = lax.axis_index("c")
    s = lax.axis_index("s")

    def zbody(i, carry):
      for u in range(DH // VL):
        rows[0][i, pl.ds(u * VL, VL)] = jnp.zeros((VL,), jnp.float32)
      return carry

    lax.fori_loop(0, K, zbody, 0)
    r0 = s * ROWS_PER_TILE
    for q in range(ROWS_PER_TILE // K):
      pltpu.sync_copy(rows[0],
                      acc.at[pl.ds(r0 + q * K, K)])
    pltpu.sync_copy(ei_hbm.at[0, pl.ds(s * EPT, EPT)],
                    src_v.at[pl.ds(0, EPT)])
    pltpu.sync_copy(ei_hbm.at[1, pl.ds(s * EPT, EPT)], dst_v)

    def transform(j):
      for u in range(K // VL):
        off = j * K + u * VL
        src_v[pl.ds(off, VL)] = src_v[pl.ds(off, VL)] * 2 + c

    transform(0)
    plsc.subcore_barrier()

    def gidx(j):
      return src_v.at[pl.ds(j * K, K)]

    def didx(j):
      return dst_v.at[pl.ds(j * K, K)]

    def wait_gather(j, v):
      pltpu.make_async_copy(z2_hbm.at[gidx(j)], rows[v], gsem[v]).wait()

    def start_scatter(j, v):
      pltpu.async_copy(rows[v], acc.at[didx(j)], ssem[v], add=True)

    def wait_scatter(j, v):
      pltpu.make_async_copy(rows[v], acc.at[didx(j)], ssem[v]).wait()

    def body(i, carry):
      j0 = NBUF * i
      for v in range(NBUF):
        j = j0 + v

        @pl.when(j >= NBUF)
        def _():
          wait_scatter(j - NBUF, v)

        pltpu.async_copy(z2_hbm.at[gidx(j)], rows[v], gsem[v])
        transform(j + 1)

        @pl.when(j >= NBUF - 1)
        def _():
          jl = j - (NBUF - 1)
          wait_gather(jl, (v + 1) % NBUF)
          start_scatter(jl, (v + 1) % NBUF)

      return carry

    lax.fori_loop(0, NCHUNK // NBUF, body, 0)
    wait_scatter(NCHUNK - NBUF, (NCHUNK - NBUF) % NBUF)
    for r in range(NCHUNK - NBUF + 1, NCHUNK):
      v = r % NBUF
      wait_gather(r, v)
      pltpu.sync_copy(rows[v], acc.at[didx(r)], add=True)

    toff = NCHUNK * K
    tsrc = src_v.at[pl.ds(toff, KTAIL)]
    tdst = dst_v.at[pl.ds(toff, KTAIL)]
    trows = rows[0].at[pl.ds(0, KTAIL)]
    pltpu.async_copy(z2_hbm.at[tsrc], trows, gsem[0]).wait()
    pltpu.sync_copy(trows, acc.at[tdst], add=True)

    plsc.subcore_barrier()

    pltpu.sync_copy(acc.at[pl.ds(r0, ROWS_PER_TILE)],
                    out_hbm.at[pl.ds(r0, ROWS_PER_TILE), pl.ds(c * DH, DH)])

  return k(z2, edge_index)


def _tc_zr_body(z_ref, wr_ref, b1_ref, o_ref):
  o_ref[...] = (jnp.dot(z_ref[...], wr_ref[...],
                        preferred_element_type=jnp.float32) + b1_ref[...])


def _tc_zr(z, W_root, b1):
  BN = 2000
  return pl.pallas_call(
      _tc_zr_body,
      grid=(N // BN,),
      in_specs=[
          pl.BlockSpec((BN, D), lambda i: (i, 0)),
          pl.BlockSpec((D, D), lambda i: (0, 0)),
          pl.BlockSpec((1, D), lambda i: (0, 0)),
      ],
      out_specs=pl.BlockSpec((BN, D), lambda i: (i, 0)),
      out_shape=jax.ShapeDtypeStruct((N, D), jnp.float32),
  )(z, W_root, b1)


def _tc_body(agg_ref, zr_ref, wn_ref, wo_ref, b2_ref, o_ref):
  h = zr_ref[...] + jnp.dot(agg_ref[...], wn_ref[...],
                            preferred_element_type=jnp.float32)
  h = jnp.maximum(h, 0.0)
  o_ref[...] = jnp.dot(h, wo_ref[...], preferred_element_type=jnp.float32) + b2_ref[...]


def _tc_decoder(agg, zr, W_nbr, W_out, b2):
  BN = 2000
  return pl.pallas_call(
      _tc_body,
      grid=(N // BN,),
      in_specs=[
          pl.BlockSpec((BN, D), lambda i: (i, 0)),
          pl.BlockSpec((BN, D), lambda i: (i, 0)),
          pl.BlockSpec((D, D), lambda i: (0, 0)),
          pl.BlockSpec((D, D), lambda i: (0, 0)),
          pl.BlockSpec((1, D), lambda i: (0, 0)),
      ],
      out_specs=pl.BlockSpec((BN, D), lambda i: (i, 0)),
      out_shape=jax.ShapeDtypeStruct((N, D), jnp.float32),
  )(agg, zr, W_nbr, W_out, b2)


def kernel(z, edge_index, W_root, W_nbr, b1, W_out, b2):
  z2 = z.reshape(2 * N, DH)
  zr = _tc_zr(z, W_root, b1.reshape(1, D))
  agg = _sc_segment_sum(z2, edge_index)
  return _tc_decoder(agg, zr, W_nbr, W_out, b2.reshape(1, D))
